# baseline (speedup 1.0000x reference)
import functools

import jax
import jax.numpy as jnp
from jax import lax
from jax.experimental import pallas as pl
from jax.experimental.pallas import tpu as pltpu
from jax.experimental.pallas import tpu_sc as plsc

PATCH = 16
TOP_K = 256
N = 1024
PD = 768
D = 96
DP = 128
RB = 256


def _tc_body(x_ref, w_ref, pb_ref, emb_ref, idx_ref, v_s, sel_s, pt_s):
    for ph in range(32):
        st3 = x_ref[0, :, ph * PATCH:(ph + 1) * PATCH, :].reshape(48, 32, PATCH)
        pt_s[:, pl.ds(ph * 32, 32)] = st3.swapaxes(1, 2).reshape(PD, 32)
    xt = pt_s[...]
    w = w_ref[...]
    emb = lax.dot_general(xt, w, (((0,), (0,)), ((), ())),
                          preferred_element_type=jnp.float32)
    emb_ref[0] = emb + pb_ref[...]
    m = jnp.sum(xt, axis=0, keepdims=True) * (1.0 / PD)
    xc = xt - m
    v = jnp.sum(xc * xc, axis=0) * (1.0 / PD)
    v_s[0, :] = v

    def rank_chunk(i, c):
        vi = v_s[0, pl.ds(i * RB, RB)][:, None]
        vrow = v_s[0, :][None, :]
        col = lax.broadcasted_iota(jnp.int32, (RB, N), 1)
        row = lax.broadcasted_iota(jnp.int32, (RB, N), 0) + i * RB
        jlt = (col < row).astype(jnp.float32)
        gt = (vrow > vi).astype(jnp.float32)
        eq = (vrow == vi).astype(jnp.float32)
        rank = jnp.sum(gt + eq * jlt, axis=1)
        sel_s[0, pl.ds(i * RB, RB)] = (rank < TOP_K).astype(jnp.float32)
        return c

    lax.fori_loop(0, N // RB, rank_chunk, 0)

    def pos_chunk(i, idx_acc):
        selc = sel_s[0, pl.ds(i * RB, RB)]
        selrow = sel_s[0, :][None, :]
        col = lax.broadcasted_iota(jnp.int32, (RB, N), 1)
        row = lax.broadcasted_iota(jnp.int32, (RB, N), 0) + i * RB
        jlt = (col < row).astype(jnp.float32)
        pos = jnp.sum(selrow * jlt, axis=1)
        kcol = lax.broadcasted_iota(jnp.int32, (RB, TOP_K), 1)
        nval = lax.broadcasted_iota(jnp.int32, (RB, TOP_K), 0) + i * RB
        oh = (pos.astype(jnp.int32)[:, None] == kcol) & (selc[:, None] > 0.0)
        return idx_acc + jnp.sum(jnp.where(oh, nval, 0), axis=0)

    idx = lax.fori_loop(0, N // RB, pos_chunk, jnp.zeros((TOP_K,), jnp.int32))
    idx_ref[0, 0] = idx + pl.program_id(0) * N


def _tc_embed(pixels, w, posb, B):
    return pl.pallas_call(
        _tc_body,
        grid=(B,),
        in_specs=[
            pl.BlockSpec((1, 3, 512, 512), lambda b: (b, 0, 0, 0)),
            pl.BlockSpec((PD, DP), lambda b: (0, 0)),
            pl.BlockSpec((N, DP), lambda b: (0, 0)),
        ],
        out_specs=[
            pl.BlockSpec((1, N, DP), lambda b: (b, 0, 0)),
            pl.BlockSpec((1, 1, TOP_K), lambda b: (b, 0, 0)),
        ],
        out_shape=[
            jax.ShapeDtypeStruct((B, N, DP), jnp.float32),
            jax.ShapeDtypeStruct((B, 1, TOP_K), jnp.int32),
        ],
        scratch_shapes=[
            pltpu.VMEM((1, N), jnp.float32),
            pltpu.VMEM((1, N), jnp.float32),
            pltpu.VMEM((PD, N), jnp.float32),
        ],
    )(pixels, w, posb)


def _sc_body(emb_hbm, idx_hbm, cls_hbm, out_hbm, idx_v, rows_v, sem):
    b = lax.axis_index("s") * 2 + lax.axis_index("c")
    pltpu.sync_copy(idx_hbm.at[b], idx_v)
    pltpu.sync_copy(cls_hbm, rows_v.at[0])
    copies = []
    for j in range(TOP_K // 16):
        iv = idx_v[pl.ds(j * 16, 16)]
        copies.append(
            pltpu.async_copy(emb_hbm.at[iv],
                             rows_v.at[pl.ds(1 + j * 16, 16)], sem))
    for c in copies:
        c.wait()
    pltpu.sync_copy(rows_v, out_hbm.at[b])


@functools.lru_cache(maxsize=None)
def _make_sc(B):
    mesh = plsc.VectorSubcoreMesh(core_axis_name="c", subcore_axis_name="s")
    return functools.partial(
        pl.kernel,
        mesh=mesh,
        out_type=jax.ShapeDtypeStruct((B, TOP_K + 1, DP), jnp.float32),
        scratch_types=[
            pltpu.VMEM((TOP_K,), jnp.int32),
            pltpu.VMEM((TOP_K + 1, DP), jnp.float32),
            pltpu.SemaphoreType.DMA,
        ],
    )(_sc_body)


def kernel(pixel_values, W_patch, b_patch, cls_token, pos_emb):
    B = pixel_values.shape[0]
    wp = jnp.pad(W_patch, ((0, 0), (0, DP - D)))
    posb = jnp.pad(pos_emb[0, 1:, :] + b_patch[None, :], ((0, 0), (0, DP - D)))
    emb, idx = _tc_embed(pixel_values, wp, posb, B)
    clsrow = jnp.pad(cls_token[0, 0] + pos_emb[0, 0], (0, DP - D))
    out = _make_sc(B)(emb.reshape(B * N, DP), idx.reshape(B, TOP_K), clsrow)
    return out[..., :D]


# speedup vs baseline: 1.0688x; 1.0688x over previous
import functools

import jax
import jax.numpy as jnp
from jax import lax
from jax.experimental import pallas as pl
from jax.experimental.pallas import tpu as pltpu
from jax.experimental.pallas import tpu_sc as plsc

PATCH = 16
TOP_K = 256
N = 1024
PD = 768
D = 96
DP = 128
RB = 256


def _tc_body(x_ref, w_ref, pb_ref, emb_ref, idx_ref, v_s, sel_s, p_s):
    for ph in range(32):
        strip = x_ref[0, :, ph * PATCH:(ph + 1) * PATCH, :]
        p_s[pl.ds(ph * 32, 32), :] = (strip.reshape(48, 32, PATCH)
                                      .transpose(1, 0, 2).reshape(32, PD))
    x = p_s[...]
    w = w_ref[...]
    emb = jnp.dot(x, w, preferred_element_type=jnp.float32)
    emb_ref[0] = emb + pb_ref[...]
    m = jnp.mean(x, axis=1, keepdims=True)
    xc = x - m
    v = jnp.sum(xc * xc, axis=1) * (1.0 / PD)
    v_s[0, :] = v

    def rank_chunk(i, c):
        vi = v_s[0, pl.ds(i * RB, RB)][:, None]
        vrow = v_s[0, :][None, :]
        col = lax.broadcasted_iota(jnp.int32, (RB, N), 1)
        row = lax.broadcasted_iota(jnp.int32, (RB, N), 0) + i * RB
        jlt = (col < row).astype(jnp.float32)
        gt = (vrow > vi).astype(jnp.float32)
        eq = (vrow == vi).astype(jnp.float32)
        rank = jnp.sum(gt + eq * jlt, axis=1)
        sel_s[0, pl.ds(i * RB, RB)] = (rank < TOP_K).astype(jnp.float32)
        return c

    lax.fori_loop(0, N // RB, rank_chunk, 0)

    def pos_chunk(i, idx_acc):
        selc = sel_s[0, pl.ds(i * RB, RB)]
        selrow = sel_s[0, :][None, :]
        col = lax.broadcasted_iota(jnp.int32, (RB, N), 1)
        row = lax.broadcasted_iota(jnp.int32, (RB, N), 0) + i * RB
        jlt = (col < row).astype(jnp.float32)
        pos = jnp.sum(selrow * jlt, axis=1)
        kcol = lax.broadcasted_iota(jnp.int32, (RB, TOP_K), 1)
        nval = lax.broadcasted_iota(jnp.int32, (RB, TOP_K), 0) + i * RB
        oh = (pos.astype(jnp.int32)[:, None] == kcol) & (selc[:, None] > 0.0)
        return idx_acc + jnp.sum(jnp.where(oh, nval, 0), axis=0)

    idx = lax.fori_loop(0, N // RB, pos_chunk, jnp.zeros((TOP_K,), jnp.int32))
    idx_ref[0, 0] = idx + pl.program_id(0) * N


def _tc_embed(pixels, w, posb, B):
    return pl.pallas_call(
        _tc_body,
        grid=(B,),
        in_specs=[
            pl.BlockSpec((1, 3, 512, 512), lambda b: (b, 0, 0, 0)),
            pl.BlockSpec((PD, DP), lambda b: (0, 0)),
            pl.BlockSpec((N, DP), lambda b: (0, 0)),
        ],
        out_specs=[
            pl.BlockSpec((1, N, DP), lambda b: (b, 0, 0)),
            pl.BlockSpec((1, 1, TOP_K), lambda b: (b, 0, 0)),
        ],
        out_shape=[
            jax.ShapeDtypeStruct((B, N, DP), jnp.float32),
            jax.ShapeDtypeStruct((B, 1, TOP_K), jnp.int32),
        ],
        scratch_shapes=[
            pltpu.VMEM((1, N), jnp.float32),
            pltpu.VMEM((1, N), jnp.float32),
            pltpu.VMEM((N, PD), jnp.float32),
        ],
        compiler_params=pltpu.CompilerParams(
            dimension_semantics=("parallel",)),
    )(pixels, w, posb)


def _sc_body(emb_hbm, idx_hbm, cls_hbm, out_hbm, idx_v, rows_v, sem):
    b = lax.axis_index("s") * 2 + lax.axis_index("c")
    pltpu.sync_copy(idx_hbm.at[b], idx_v)
    pltpu.sync_copy(cls_hbm, rows_v.at[0])
    copies = []
    for j in range(TOP_K // 16):
        iv = idx_v[pl.ds(j * 16, 16)]
        copies.append(
            pltpu.async_copy(emb_hbm.at[iv],
                             rows_v.at[pl.ds(1 + j * 16, 16)], sem))
    for c in copies:
        c.wait()
    pltpu.sync_copy(rows_v, out_hbm.at[b])


@functools.lru_cache(maxsize=None)
def _make_sc(B):
    mesh = plsc.VectorSubcoreMesh(core_axis_name="c", subcore_axis_name="s")
    return functools.partial(
        pl.kernel,
        mesh=mesh,
        out_type=jax.ShapeDtypeStruct((B, TOP_K + 1, DP), jnp.float32),
        scratch_types=[
            pltpu.VMEM((TOP_K,), jnp.int32),
            pltpu.VMEM((TOP_K + 1, DP), jnp.float32),
            pltpu.SemaphoreType.DMA,
        ],
    )(_sc_body)


def kernel(pixel_values, W_patch, b_patch, cls_token, pos_emb):
    B = pixel_values.shape[0]
    wp = jnp.pad(W_patch, ((0, 0), (0, DP - D)))
    posb = jnp.pad(pos_emb[0, 1:, :] + b_patch[None, :], ((0, 0), (0, DP - D)))
    emb, idx = _tc_embed(pixel_values, wp, posb, B)
    clsrow = jnp.pad(cls_token[0, 0] + pos_emb[0, 0], (0, DP - D))
    out = _make_sc(B)(emb.reshape(B * N, DP), idx.reshape(B, TOP_K), clsrow)
    return out[..., :D]
